# baseline (speedup 1.0000x reference)
import functools

import jax
import jax.numpy as jnp
from jax import lax
from jax.experimental import pallas as pl
from jax.experimental.pallas import tpu as pltpu
from jax.experimental.pallas import tpu_sc as plsc

B = 128
L = 512
D = 768
LANES = 16
NW = 32
LPW = L // NW
DV = D // LANES


def _embed(tok_hbm, word_hbm, pos_hbm, out_hbm, idx_v, pos_v, rows_v, sem):
    wid = lax.axis_index("s") * 2 + lax.axis_index("c")
    l0 = wid * LPW

    pltpu.sync_copy(tok_hbm, idx_v)
    pltpu.sync_copy(pos_hbm.at[pl.ds(l0, LPW)], pos_v)

    def body(b, carry):
        pltpu.async_copy(word_hbm.at[idx_v.at[b, pl.ds(l0, LPW)]], rows_v, sem).wait()

        def add_row(i, c):
            for j in range(DV):
                sl = pl.ds(j * LANES, LANES)
                plsc.addupdate(rows_v.at[i, sl], pos_v[i, sl])
            return c

        lax.fori_loop(0, LPW, add_row, None)

        pltpu.sync_copy(rows_v, out_hbm.at[b, pl.ds(l0, LPW)])
        return carry

    lax.fori_loop(0, B, body, None)


def kernel(input_tokens, word_emb, pos_emb):
    mesh = plsc.VectorSubcoreMesh(core_axis_name="c", subcore_axis_name="s")
    run = functools.partial(
        pl.kernel,
        out_type=jax.ShapeDtypeStruct((B, L, D), jnp.float32),
        mesh=mesh,
        scratch_types=[
            pltpu.VMEM((B, L), jnp.int32),
            pltpu.VMEM((LPW, D), jnp.float32),
            pltpu.VMEM((LPW, D), jnp.float32),
            pltpu.SemaphoreType.DMA,
        ],
    )(_embed)
    return run(input_tokens, word_emb, pos_emb)


# speedup vs baseline: 1.4897x; 1.4897x over previous
import functools

import jax
import jax.numpy as jnp
from jax import lax
from jax.experimental import pallas as pl
from jax.experimental.pallas import tpu as pltpu
from jax.experimental.pallas import tpu_sc as plsc

B = 128
L = 512
D = 768
LANES = 16
NW = 32
LPW = L // NW
DV = D // LANES
NBUF = 4
G = B // NBUF


def _embed(tok_hbm, word_hbm, pos_hbm, out_hbm,
           idx_v, pos_v, gbuf, sbuf, gsem, ssem):
    wid = lax.axis_index("s") * 2 + lax.axis_index("c")
    l0 = wid * LPW

    pltpu.sync_copy(tok_hbm.at[wid], idx_v)
    pltpu.sync_copy(pos_hbm.at[pl.ds(l0, LPW)], pos_v)

    for k in range(NBUF):
        pltpu.async_copy(word_hbm.at[idx_v.at[k]], gbuf.at[k], gsem.at[k])

    def outer(g, carry):
        for k in range(NBUF):
            b = g * NBUF + k

            @pl.when(g > 0)
            def _():
                pltpu.make_async_copy(
                    sbuf.at[k], out_hbm.at[b, pl.ds(l0, LPW)], ssem.at[k]
                ).wait()

            pltpu.make_async_copy(
                word_hbm.at[idx_v.at[b]], gbuf.at[k], gsem.at[k]
            ).wait()

            def add_row(i, c):
                for j in range(DV):
                    sl = pl.ds(j * LANES, LANES)
                    sbuf[k, i, sl] = gbuf[k, i, sl] + pos_v[i, sl]
                return c

            lax.fori_loop(0, LPW, add_row, None)

            @pl.when(g < G - 1)
            def _():
                pltpu.async_copy(
                    word_hbm.at[idx_v.at[b + NBUF]], gbuf.at[k], gsem.at[k]
                )

            pltpu.async_copy(
                sbuf.at[k], out_hbm.at[b, pl.ds(l0, LPW)], ssem.at[k]
            )
        return carry

    lax.fori_loop(0, G, outer, None)

    for k in range(NBUF):
        b = (G - 1) * NBUF + k
        pltpu.make_async_copy(
            sbuf.at[k], out_hbm.at[b, pl.ds(l0, LPW)], ssem.at[k]
        ).wait()


def kernel(input_tokens, word_emb, pos_emb):
    tok_arr = jnp.transpose(input_tokens.reshape(B, NW, LPW), (1, 0, 2))
    mesh = plsc.VectorSubcoreMesh(core_axis_name="c", subcore_axis_name="s")
    run = functools.partial(
        pl.kernel,
        out_type=jax.ShapeDtypeStruct((B, L, D), jnp.float32),
        mesh=mesh,
        scratch_types=[
            pltpu.VMEM((B, LPW), jnp.int32),
            pltpu.VMEM((LPW, D), jnp.float32),
            pltpu.VMEM((NBUF, LPW, D), jnp.float32),
            pltpu.VMEM((NBUF, LPW, D), jnp.float32),
            pltpu.SemaphoreType.DMA((NBUF,)),
            pltpu.SemaphoreType.DMA((NBUF,)),
        ],
    )(_embed)
    return run(tok_arr, word_emb, pos_emb)
